# baseline (speedup 1.0000x reference)
import functools

import jax
import jax.numpy as jnp
from jax import lax
from jax.experimental import pallas as pl
from jax.experimental.pallas import tpu as pltpu
from jax.experimental.pallas import tpu_sc as plsc

_N = 10000
_E = 320000
_D = 128
_NC = 2
_NS = 16
_NW = _NC * _NS
_C = 64
_J = 162
_EPW = _J * _C
_JW = _EPW // 128
_EP = _NW * _EPW
_NP = 10240
_RPT = _NP // _NS
_AR = _NP
_ZR = _RPT
_G = 3
_IPW = 80
_IP = _NW * _IPW

_MESH = plsc.VectorSubcoreMesh(
    core_axis_name="c", subcore_axis_name="s", num_cores=_NC, num_subcores=_NS
)


def _stage_idx(dst1d, src2d, g):
    r = g // (128 // _C)
    cb = (g % (128 // _C)) * _C
    for k in range(_C // 16):
        dst1d[pl.ds(k * 16, 16)] = src2d[r, pl.ds(cb + k * 16, 16)]


def _deg_body(dstp, ones_hbm, zeros_hbm, deg, dst_v, didx, ones_v, acc):
    c = lax.axis_index("c")
    s = lax.axis_index("s")
    wid = c * _NS + s
    pltpu.sync_copy(dstp.at[wid], dst_v)
    pltpu.sync_copy(ones_hbm, ones_v)
    pltpu.sync_copy(zeros_hbm, acc.at[pl.ds(s * _ZR, _ZR)])
    plsc.subcore_barrier()

    def body(j, carry):
        _stage_idx(didx, dst_v, j)
        pltpu.sync_copy(ones_v, acc.at[didx], add=True)
        return carry

    lax.fori_loop(0, _J, body, 0)
    plsc.subcore_barrier()
    pltpu.sync_copy(acc.at[pl.ds(s * _RPT, _RPT)], deg.at[c, pl.ds(s * _RPT, _RPT)])


_deg_call = pl.kernel(
    _deg_body,
    out_type=jax.ShapeDtypeStruct((_NC, _NP, _D), jnp.float32),
    mesh=_MESH,
    scratch_types=[
        pltpu.VMEM((_JW, 128), jnp.int32),
        pltpu.VMEM((_C,), jnp.int32),
        pltpu.VMEM((_C, _D), jnp.float32),
        pltpu.VMEM_SHARED((_AR, _D), jnp.float32),
    ],
)


def _prop_body(y_hbm, srcp, dstp, out,
               src_v, dst_v,
               sidx0, sidx1, sidx2, didx0, didx1, didx2,
               rows0, rows1, rows2, acc,
               gsem0, gsem1, gsem2, ssem0, ssem1, ssem2):
    c = lax.axis_index("c")
    s = lax.axis_index("s")
    wid = c * _NS + s
    pltpu.sync_copy(srcp.at[wid], src_v)
    pltpu.sync_copy(dstp.at[wid], dst_v)

    pltpu.sync_copy(y_hbm.at[pl.ds(s * _RPT, _RPT)], acc.at[pl.ds(s * _RPT, _RPT)])
    plsc.subcore_barrier()

    sidx = (sidx0, sidx1, sidx2)
    didx = (didx0, didx1, didx2)
    rows = (rows0, rows1, rows2)
    gsem = (gsem0, gsem1, gsem2)
    ssem = (ssem0, ssem1, ssem2)

    def group(t, carry):
        for b in range(_G):
            g = t * _G + b

            @pl.when(t > 0)
            def _():
                pltpu.make_async_copy(rows[b], acc.at[didx[b]], ssem[b]).wait()

            _stage_idx(sidx[b], src_v, g)
            _stage_idx(didx[b], dst_v, g)
            pltpu.async_copy(y_hbm.at[sidx[b]], rows[b], gsem[b])
        for b in range(_G):
            pltpu.make_async_copy(y_hbm.at[sidx[b]], rows[b], gsem[b]).wait()
            pltpu.async_copy(rows[b], acc.at[didx[b]], ssem[b], add=True)
        return carry

    lax.fori_loop(0, _J // _G, group, 0)
    for b in range(_G):
        pltpu.make_async_copy(rows[b], acc.at[didx[b]], ssem[b]).wait()
    plsc.subcore_barrier()
    pltpu.sync_copy(acc.at[pl.ds(s * _RPT, _RPT)], out.at[c, pl.ds(s * _RPT, _RPT)])


_prop_call = pl.kernel(
    _prop_body,
    out_type=jax.ShapeDtypeStruct((_NC, _NP, _D), jnp.float32),
    mesh=_MESH,
    scratch_types=(
        [pltpu.VMEM((_JW, 128), jnp.int32)] * 2
        + [pltpu.VMEM((_C,), jnp.int32)] * (2 * _G)
        + [pltpu.VMEM((_C, _D), jnp.float32)] * _G
        + [pltpu.VMEM_SHARED((_AR, _D), jnp.float32)]
        + [pltpu.SemaphoreType.DMA] * (2 * _G)
    ),
)


def _gat_body(h_hbm, pb_hbm, idxp, hout, pbout, idx_v, hrows, prows, sem):
    c = lax.axis_index("c")
    s = lax.axis_index("s")
    wid = c * _NS + s
    pltpu.sync_copy(idxp.at[wid], idx_v)
    pltpu.async_copy(h_hbm.at[idx_v], hrows, sem).wait()
    pltpu.async_copy(pb_hbm.at[idx_v], prows, sem).wait()
    pltpu.sync_copy(hrows, hout.at[pl.ds(wid * _IPW, _IPW)])
    pltpu.sync_copy(prows, pbout.at[pl.ds(wid * _IPW, _IPW)])


_gat_call = pl.kernel(
    _gat_body,
    out_type=(
        jax.ShapeDtypeStruct((_IP, _D), jnp.float32),
        jax.ShapeDtypeStruct((_IP, _D), jnp.float32),
    ),
    mesh=_MESH,
    scratch_types=[
        pltpu.VMEM((_IPW,), jnp.int32),
        pltpu.VMEM((_IPW, _D), jnp.float32),
        pltpu.VMEM((_IPW, _D), jnp.float32),
        pltpu.SemaphoreType.DMA,
    ],
)


_BR = 1024


def _stage1_body(deg, x, pos8, w1x, w1p, y, dinv):
    d2 = deg[...]
    dg = d2[0][:, 0:1] + d2[1][:, 0:1] + 1.0
    di = lax.rsqrt(dg)
    h = jnp.dot(x[...], w1x[...], preferred_element_type=jnp.float32,
                precision=lax.Precision.HIGHEST)
    h = h + jnp.dot(pos8[...], w1p[...], preferred_element_type=jnp.float32,
                    precision=lax.Precision.HIGHEST)
    y[...] = h * di
    dinv[...] = di


_stage1_call = pl.pallas_call(
    _stage1_body,
    grid=(_NP // _BR,),
    in_specs=[
        pl.BlockSpec((_NC, _BR, _D), lambda i: (0, i, 0)),
        pl.BlockSpec((_BR, _D), lambda i: (i, 0)),
        pl.BlockSpec((_BR, 8), lambda i: (i, 0)),
        pl.BlockSpec((_D, _D), lambda i: (0, 0)),
        pl.BlockSpec((8, _D), lambda i: (0, 0)),
    ],
    out_specs=[
        pl.BlockSpec((_BR, _D), lambda i: (i, 0)),
        pl.BlockSpec((_BR, 1), lambda i: (i, 0)),
    ],
    out_shape=[
        jax.ShapeDtypeStruct((_NP, _D), jnp.float32),
        jax.ShapeDtypeStruct((_NP, 1), jnp.float32),
    ],
)


def _mid_body(p, yin, dinv, w, b, y):
    p2 = p[...]
    di = dinv[...]
    h = jnp.maximum((p2[0] + p2[1] - yin[...]) * di + b[...], 0.0)
    y[...] = jnp.dot(h, w[...], preferred_element_type=jnp.float32,
                     precision=lax.Precision.HIGHEST) * di


_mid_call = pl.pallas_call(
    _mid_body,
    grid=(_NP // _BR,),
    in_specs=[
        pl.BlockSpec((_NC, _BR, _D), lambda i: (0, i, 0)),
        pl.BlockSpec((_BR, _D), lambda i: (i, 0)),
        pl.BlockSpec((_BR, 1), lambda i: (i, 0)),
        pl.BlockSpec((_D, _D), lambda i: (0, 0)),
        pl.BlockSpec((1, _D), lambda i: (0, 0)),
    ],
    out_specs=pl.BlockSpec((_BR, _D), lambda i: (i, 0)),
    out_shape=jax.ShapeDtypeStruct((_NP, _D), jnp.float32),
)


def _last_body(p, yin, dinv, b, h):
    p2 = p[...]
    h[...] = jnp.maximum((p2[0] + p2[1] - yin[...]) * dinv[...] + b[...], 0.0)


_last_call = pl.pallas_call(
    _last_body,
    grid=(_NP // _BR,),
    in_specs=[
        pl.BlockSpec((_NC, _BR, _D), lambda i: (0, i, 0)),
        pl.BlockSpec((_BR, _D), lambda i: (i, 0)),
        pl.BlockSpec((_BR, 1), lambda i: (i, 0)),
        pl.BlockSpec((1, _D), lambda i: (0, 0)),
    ],
    out_specs=pl.BlockSpec((_BR, _D), lambda i: (i, 0)),
    out_shape=jax.ShapeDtypeStruct((_NP, _D), jnp.float32),
)


def kernel(x, pos, batch, idx, edge_index, W1, b1, W2, b2, W3, b3):
    m = idx.shape[0]
    src = edge_index[0].astype(jnp.int32)
    dst = edge_index[1].astype(jnp.int32)
    srcp = jnp.pad(src, (0, _EP - _E)).reshape(_NW, _JW, 128)
    dstp = jnp.pad(dst, (0, _EP - _E), constant_values=_N).reshape(_NW, _JW, 128)
    zeros128 = jnp.zeros((_ZR, _D), jnp.float32)
    ones128 = jnp.ones((_C, _D), jnp.float32)
    xp = jnp.pad(x, ((0, _NP - _N), (0, 0)))
    pos8 = jnp.pad(pos, ((0, _NP - _N), (0, 5)))
    w1x = W1[:_D]
    w1p = jnp.pad(W1[_D:], ((0, 5), (0, 0)))
    posb = jnp.concatenate(
        [pos, batch.astype(jnp.float32)[:, None],
         jnp.zeros((_N, _D - 4), jnp.float32)], axis=1)
    idxp = jnp.pad(idx.astype(jnp.int32), (0, _IP - m)).reshape(_NW, _IPW)
    b1r = b1.reshape(1, _D)
    b2r = b2.reshape(1, _D)
    b3r = b3.reshape(1, _D)

    deg = _deg_call(dstp, ones128, zeros128)
    y1, dinv = _stage1_call(deg, xp, pos8, w1x, w1p)
    p = _prop_call(y1, srcp, dstp)
    y2 = _mid_call(p, y1, dinv, W2, b1r)
    p = _prop_call(y2, srcp, dstp)
    y3 = _mid_call(p, y2, dinv, W3, b2r)
    p = _prop_call(y3, srcp, dstp)
    h3 = _last_call(p, y3, dinv, b3r)
    hout, pbout = _gat_call(h3, posb, idxp)
    return (
        hout[:m],
        pbout[:m, :3],
        pbout[:m, 3].astype(jnp.int32),
    )


# speedup vs baseline: 1.3038x; 1.3038x over previous
import functools

import jax
import jax.numpy as jnp
from jax import lax
from jax.experimental import pallas as pl
from jax.experimental.pallas import tpu as pltpu
from jax.experimental.pallas import tpu_sc as plsc

_N = 10000
_E = 320000
_D = 128
_NC = 2
_NS = 16
_NW = _NC * _NS
_C = 128
_J = 80
_EPW = _J * _C
_JW = _EPW // 128
_EP = _NW * _EPW
_NP = 10240
_RPT = _NP // _NS
_AR = _NP
_ZR = _RPT
_G = 2
_IPW = 80
_IP = _NW * _IPW

_MESH = plsc.VectorSubcoreMesh(
    core_axis_name="c", subcore_axis_name="s", num_cores=_NC, num_subcores=_NS
)


def _stage_idx(dst1d, src2d, g):
    r = g // (128 // _C)
    cb = (g % (128 // _C)) * _C
    for k in range(_C // 16):
        dst1d[pl.ds(k * 16, 16)] = src2d[r, pl.ds(cb + k * 16, 16)]


def _deg_body(dstp, ones_hbm, zeros_hbm, deg, dst_v, didx, ones_v, acc):
    c = lax.axis_index("c")
    s = lax.axis_index("s")
    wid = c * _NS + s
    pltpu.sync_copy(dstp.at[wid], dst_v)
    pltpu.sync_copy(ones_hbm, ones_v)
    pltpu.sync_copy(zeros_hbm, acc.at[pl.ds(s * _ZR, _ZR)])
    plsc.subcore_barrier()

    def body(j, carry):
        _stage_idx(didx, dst_v, j)
        pltpu.sync_copy(ones_v, acc.at[didx], add=True)
        return carry

    lax.fori_loop(0, _J, body, 0)
    plsc.subcore_barrier()
    pltpu.sync_copy(acc.at[pl.ds(s * _RPT, _RPT)], deg.at[c, pl.ds(s * _RPT, _RPT)])


_deg_call = pl.kernel(
    _deg_body,
    out_type=jax.ShapeDtypeStruct((_NC, _NP, _D), jnp.float32),
    mesh=_MESH,
    scratch_types=[
        pltpu.VMEM((_JW, 128), jnp.int32),
        pltpu.VMEM((_C,), jnp.int32),
        pltpu.VMEM((_C, _D), jnp.float32),
        pltpu.VMEM_SHARED((_AR, _D), jnp.float32),
    ],
)


def _prop_body(y_hbm, srcf, dstp, out,
               dst_v, sidx0, sidx1, didx0, didx1, rows0, rows1, acc,
               isem0, isem1, gsem0, gsem1, ssem0, ssem1):
    c = lax.axis_index("c")
    s = lax.axis_index("s")
    wid = c * _NS + s
    pltpu.sync_copy(dstp.at[wid], dst_v)
    pltpu.sync_copy(y_hbm.at[pl.ds(s * _RPT, _RPT)], acc.at[pl.ds(s * _RPT, _RPT)])
    sidx = (sidx0, sidx1)
    didx = (didx0, didx1)
    rows = (rows0, rows1)
    isem = (isem0, isem1)
    gsem = (gsem0, gsem1)
    ssem = (ssem0, ssem1)
    base = wid * _J
    for b in range(2):
        pltpu.async_copy(srcf.at[base + b], sidx[b], isem[b])
    plsc.subcore_barrier()

    def group(t, carry):
        for b in range(2):
            @pl.when(t > 0)
            def _():
                pltpu.make_async_copy(rows[b], acc.at[didx[b]], ssem[b]).wait()

            _stage_idx(didx[b], dst_v, 2 * t + b)
        for b in range(2):
            pltpu.make_async_copy(srcf.at[base + 2 * t + b], sidx[b], isem[b]).wait()
            pltpu.async_copy(y_hbm.at[sidx[b]], rows[b], gsem[b])
        for b in range(2):
            g = 2 * t + b
            pltpu.make_async_copy(y_hbm.at[sidx[b]], rows[b], gsem[b]).wait()
            pltpu.async_copy(rows[b], acc.at[didx[b]], ssem[b], add=True)

            @pl.when(g + 2 < _J)
            def _():
                pltpu.async_copy(srcf.at[base + g + 2], sidx[b], isem[b])

        return carry

    lax.fori_loop(0, _J // 2, group, 0)
    for b in range(2):
        pltpu.make_async_copy(rows[b], acc.at[didx[b]], ssem[b]).wait()
    plsc.subcore_barrier()
    pltpu.sync_copy(acc.at[pl.ds(s * _RPT, _RPT)], out.at[c, pl.ds(s * _RPT, _RPT)])


_prop_call = pl.kernel(
    _prop_body,
    out_type=jax.ShapeDtypeStruct((_NC, _NP, _D), jnp.float32),
    mesh=_MESH,
    scratch_types=(
        [pltpu.VMEM((_JW, 128), jnp.int32)]
        + [pltpu.VMEM((_C,), jnp.int32)] * 4
        + [pltpu.VMEM((_C, _D), jnp.float32)] * 2
        + [pltpu.VMEM_SHARED((_AR, _D), jnp.float32)]
        + [pltpu.SemaphoreType.DMA] * 6
    ),
)


def _gat_body(h_hbm, pb_hbm, idxp, hout, pbout, idx_v, hrows, prows, sem):
    c = lax.axis_index("c")
    s = lax.axis_index("s")
    wid = c * _NS + s
    pltpu.sync_copy(idxp.at[wid], idx_v)
    pltpu.async_copy(h_hbm.at[idx_v], hrows, sem).wait()
    pltpu.async_copy(pb_hbm.at[idx_v], prows, sem).wait()
    pltpu.sync_copy(hrows, hout.at[pl.ds(wid * _IPW, _IPW)])
    pltpu.sync_copy(prows, pbout.at[pl.ds(wid * _IPW, _IPW)])


_gat_call = pl.kernel(
    _gat_body,
    out_type=(
        jax.ShapeDtypeStruct((_IP, _D), jnp.float32),
        jax.ShapeDtypeStruct((_IP, _D), jnp.float32),
    ),
    mesh=_MESH,
    scratch_types=[
        pltpu.VMEM((_IPW,), jnp.int32),
        pltpu.VMEM((_IPW, _D), jnp.float32),
        pltpu.VMEM((_IPW, _D), jnp.float32),
        pltpu.SemaphoreType.DMA,
    ],
)


_BR = 1024


def _stage1_body(deg, x, pos8, w1x, w1p, y, dinv):
    d2 = deg[...]
    dg = d2[0][:, 0:1] + d2[1][:, 0:1] + 1.0
    di = lax.rsqrt(dg)
    h = jnp.dot(x[...], w1x[...], preferred_element_type=jnp.float32,
                precision=lax.Precision.HIGHEST)
    h = h + jnp.dot(pos8[...], w1p[...], preferred_element_type=jnp.float32,
                    precision=lax.Precision.HIGHEST)
    y[...] = h * di
    dinv[...] = di


_stage1_call = pl.pallas_call(
    _stage1_body,
    grid=(_NP // _BR,),
    in_specs=[
        pl.BlockSpec((_NC, _BR, _D), lambda i: (0, i, 0)),
        pl.BlockSpec((_BR, _D), lambda i: (i, 0)),
        pl.BlockSpec((_BR, 8), lambda i: (i, 0)),
        pl.BlockSpec((_D, _D), lambda i: (0, 0)),
        pl.BlockSpec((8, _D), lambda i: (0, 0)),
    ],
    out_specs=[
        pl.BlockSpec((_BR, _D), lambda i: (i, 0)),
        pl.BlockSpec((_BR, 1), lambda i: (i, 0)),
    ],
    out_shape=[
        jax.ShapeDtypeStruct((_NP, _D), jnp.float32),
        jax.ShapeDtypeStruct((_NP, 1), jnp.float32),
    ],
)


def _mid_body(p, yin, dinv, w, b, y):
    p2 = p[...]
    di = dinv[...]
    h = jnp.maximum((p2[0] + p2[1] - yin[...]) * di + b[...], 0.0)
    y[...] = jnp.dot(h, w[...], preferred_element_type=jnp.float32,
                     precision=lax.Precision.HIGHEST) * di


_mid_call = pl.pallas_call(
    _mid_body,
    grid=(_NP // _BR,),
    in_specs=[
        pl.BlockSpec((_NC, _BR, _D), lambda i: (0, i, 0)),
        pl.BlockSpec((_BR, _D), lambda i: (i, 0)),
        pl.BlockSpec((_BR, 1), lambda i: (i, 0)),
        pl.BlockSpec((_D, _D), lambda i: (0, 0)),
        pl.BlockSpec((1, _D), lambda i: (0, 0)),
    ],
    out_specs=pl.BlockSpec((_BR, _D), lambda i: (i, 0)),
    out_shape=jax.ShapeDtypeStruct((_NP, _D), jnp.float32),
)


def _last_body(p, yin, dinv, b, h):
    p2 = p[...]
    h[...] = jnp.maximum((p2[0] + p2[1] - yin[...]) * dinv[...] + b[...], 0.0)


_last_call = pl.pallas_call(
    _last_body,
    grid=(_NP // _BR,),
    in_specs=[
        pl.BlockSpec((_NC, _BR, _D), lambda i: (0, i, 0)),
        pl.BlockSpec((_BR, _D), lambda i: (i, 0)),
        pl.BlockSpec((_BR, 1), lambda i: (i, 0)),
        pl.BlockSpec((1, _D), lambda i: (0, 0)),
    ],
    out_specs=pl.BlockSpec((_BR, _D), lambda i: (i, 0)),
    out_shape=jax.ShapeDtypeStruct((_NP, _D), jnp.float32),
)


def kernel(x, pos, batch, idx, edge_index, W1, b1, W2, b2, W3, b3):
    m = idx.shape[0]
    src = edge_index[0].astype(jnp.int32)
    dst = edge_index[1].astype(jnp.int32)
    srcf = jnp.pad(src, (0, _EP - _E)).reshape(_NW * _J, _C)
    dstp = jnp.pad(dst, (0, _EP - _E), constant_values=_N).reshape(_NW, _JW, 128)
    zeros128 = jnp.zeros((_ZR, _D), jnp.float32)
    ones128 = jnp.ones((_C, _D), jnp.float32)
    xp = jnp.pad(x, ((0, _NP - _N), (0, 0)))
    pos8 = jnp.pad(pos, ((0, _NP - _N), (0, 5)))
    w1x = W1[:_D]
    w1p = jnp.pad(W1[_D:], ((0, 5), (0, 0)))
    posb = jnp.concatenate(
        [pos, batch.astype(jnp.float32)[:, None],
         jnp.zeros((_N, _D - 4), jnp.float32)], axis=1)
    idxp = jnp.pad(idx.astype(jnp.int32), (0, _IP - m)).reshape(_NW, _IPW)
    b1r = b1.reshape(1, _D)
    b2r = b2.reshape(1, _D)
    b3r = b3.reshape(1, _D)

    deg = _deg_call(dstp, ones128, zeros128)
    y1, dinv = _stage1_call(deg, xp, pos8, w1x, w1p)
    p = _prop_call(y1, srcf, dstp)
    y2 = _mid_call(p, y1, dinv, W2, b1r)
    p = _prop_call(y2, srcf, dstp)
    y3 = _mid_call(p, y2, dinv, W3, b2r)
    p = _prop_call(y3, srcf, dstp)
    h3 = _last_call(p, y3, dinv, b3r)
    hout, pbout = _gat_call(h3, posb, idxp)
    return (
        hout[:m],
        pbout[:m, :3],
        pbout[:m, 3].astype(jnp.int32),
    )


# speedup vs baseline: 1.3138x; 1.0077x over previous
import functools

import jax
import jax.numpy as jnp
from jax import lax
from jax.experimental import pallas as pl
from jax.experimental.pallas import tpu as pltpu
from jax.experimental.pallas import tpu_sc as plsc

_N = 10000
_E = 320000
_D = 128
_NC = 2
_NS = 16
_NW = _NC * _NS
_C = 128
_J = 80
_EPW = _J * _C
_JW = _EPW // 128
_EP = _NW * _EPW
_NP = 10240
_RPT = _NP // _NS
_AR = _NP
_ZR = _RPT
_G = 2
_J0 = 40
_J1 = 120
_JMX = 120
_IPW = 80
_IP = _NW * _IPW

_MESH = plsc.VectorSubcoreMesh(
    core_axis_name="c", subcore_axis_name="s", num_cores=_NC, num_subcores=_NS
)


def _stage_idx(dst1d, src2d, g):
    r = g // (128 // _C)
    cb = (g % (128 // _C)) * _C
    for k in range(_C // 16):
        dst1d[pl.ds(k * 16, 16)] = src2d[r, pl.ds(cb + k * 16, 16)]


def _deg_body(dstp, ones_hbm, zeros_hbm, deg, dst_v, didx, ones_v, acc):
    c = lax.axis_index("c")
    s = lax.axis_index("s")
    wid = c * _NS + s
    pltpu.sync_copy(dstp.at[wid], dst_v)
    pltpu.sync_copy(ones_hbm, ones_v)
    pltpu.sync_copy(zeros_hbm, acc.at[pl.ds(s * _ZR, _ZR)])
    plsc.subcore_barrier()

    def body(j, carry):
        _stage_idx(didx, dst_v, j)
        pltpu.sync_copy(ones_v, acc.at[didx], add=True)
        return carry

    lax.fori_loop(0, _J, body, 0)
    plsc.subcore_barrier()
    pltpu.sync_copy(acc.at[pl.ds(s * _RPT, _RPT)], deg.at[c, pl.ds(s * _RPT, _RPT)])


_deg_call = pl.kernel(
    _deg_body,
    out_type=jax.ShapeDtypeStruct((_NC, _NP, _D), jnp.float32),
    mesh=_MESH,
    scratch_types=[
        pltpu.VMEM((_JW, 128), jnp.int32),
        pltpu.VMEM((_C,), jnp.int32),
        pltpu.VMEM((_C, _D), jnp.float32),
        pltpu.VMEM_SHARED((_AR, _D), jnp.float32),
    ],
)


def _prop_body(y_hbm, srcf, dstf, out,
               dst_v, sidx0, sidx1, didx0, didx1, rows0, rows1, acc,
               isem0, isem1, gsem0, gsem1, ssem0, ssem1):
    c = lax.axis_index("c")
    s = lax.axis_index("s")
    jn = lax.select(c == 0, _J0, _J1)
    base = lax.select(c == 0, s * _J0, _NS * _J0 + s * _J1)
    pltpu.sync_copy(dstf.at[pl.ds(base, _JMX)], dst_v)
    pltpu.sync_copy(y_hbm.at[pl.ds(s * _RPT, _RPT)], acc.at[pl.ds(s * _RPT, _RPT)])
    sidx = (sidx0, sidx1)
    didx = (didx0, didx1)
    rows = (rows0, rows1)
    isem = (isem0, isem1)
    gsem = (gsem0, gsem1)
    ssem = (ssem0, ssem1)
    for b in range(2):
        pltpu.async_copy(srcf.at[base + b], sidx[b], isem[b])
    plsc.subcore_barrier()

    def group(t, carry):
        for b in range(2):
            @pl.when(t > 0)
            def _():
                pltpu.make_async_copy(rows[b], acc.at[didx[b]], ssem[b]).wait()

            _stage_idx(didx[b], dst_v, 2 * t + b)
        for b in range(2):
            pltpu.make_async_copy(srcf.at[base + 2 * t + b], sidx[b], isem[b]).wait()
            pltpu.async_copy(y_hbm.at[sidx[b]], rows[b], gsem[b])
        for b in range(2):
            g = 2 * t + b
            pltpu.make_async_copy(y_hbm.at[sidx[b]], rows[b], gsem[b]).wait()
            pltpu.async_copy(rows[b], acc.at[didx[b]], ssem[b], add=True)

            @pl.when(g + 2 < jn)
            def _():
                pltpu.async_copy(srcf.at[base + g + 2], sidx[b], isem[b])

        return carry

    lax.fori_loop(0, jn // 2, group, 0)
    for b in range(2):
        pltpu.make_async_copy(rows[b], acc.at[didx[b]], ssem[b]).wait()
    plsc.subcore_barrier()
    pltpu.sync_copy(acc.at[pl.ds(s * _RPT, _RPT)], out.at[c, pl.ds(s * _RPT, _RPT)])


_prop_call = pl.kernel(
    _prop_body,
    out_type=jax.ShapeDtypeStruct((_NC, _NP, _D), jnp.float32),
    mesh=_MESH,
    scratch_types=(
        [pltpu.VMEM((_JMX, 128), jnp.int32)]
        + [pltpu.VMEM((_C,), jnp.int32)] * 4
        + [pltpu.VMEM((_C, _D), jnp.float32)] * 2
        + [pltpu.VMEM_SHARED((_AR, _D), jnp.float32)]
        + [pltpu.SemaphoreType.DMA] * 6
    ),
)


def _gat_body(h_hbm, pb_hbm, idxp, hout, pbout, idx_v, hrows, prows, sem):
    c = lax.axis_index("c")
    s = lax.axis_index("s")
    wid = c * _NS + s
    pltpu.sync_copy(idxp.at[wid], idx_v)
    pltpu.async_copy(h_hbm.at[idx_v], hrows, sem).wait()
    pltpu.async_copy(pb_hbm.at[idx_v], prows, sem).wait()
    pltpu.sync_copy(hrows, hout.at[pl.ds(wid * _IPW, _IPW)])
    pltpu.sync_copy(prows, pbout.at[pl.ds(wid * _IPW, _IPW)])


_gat_call = pl.kernel(
    _gat_body,
    out_type=(
        jax.ShapeDtypeStruct((_IP, _D), jnp.float32),
        jax.ShapeDtypeStruct((_IP, _D), jnp.float32),
    ),
    mesh=_MESH,
    scratch_types=[
        pltpu.VMEM((_IPW,), jnp.int32),
        pltpu.VMEM((_IPW, _D), jnp.float32),
        pltpu.VMEM((_IPW, _D), jnp.float32),
        pltpu.SemaphoreType.DMA,
    ],
)


_BR = 1024


def _stage1_body(deg, x, pos8, w1x, w1p, y, dinv):
    d2 = deg[...]
    dg = d2[0][:, 0:1] + d2[1][:, 0:1] + 1.0
    di = lax.rsqrt(dg)
    h = jnp.dot(x[...], w1x[...], preferred_element_type=jnp.float32,
                precision=lax.Precision.HIGHEST)
    h = h + jnp.dot(pos8[...], w1p[...], preferred_element_type=jnp.float32,
                    precision=lax.Precision.HIGHEST)
    y[...] = h * di
    dinv[...] = di


_stage1_call = pl.pallas_call(
    _stage1_body,
    grid=(_NP // _BR,),
    in_specs=[
        pl.BlockSpec((_NC, _BR, _D), lambda i: (0, i, 0)),
        pl.BlockSpec((_BR, _D), lambda i: (i, 0)),
        pl.BlockSpec((_BR, 8), lambda i: (i, 0)),
        pl.BlockSpec((_D, _D), lambda i: (0, 0)),
        pl.BlockSpec((8, _D), lambda i: (0, 0)),
    ],
    out_specs=[
        pl.BlockSpec((_BR, _D), lambda i: (i, 0)),
        pl.BlockSpec((_BR, 1), lambda i: (i, 0)),
    ],
    out_shape=[
        jax.ShapeDtypeStruct((_NP, _D), jnp.float32),
        jax.ShapeDtypeStruct((_NP, 1), jnp.float32),
    ],
)


def _mid_body(p, yin, dinv, w, b, y):
    p2 = p[...]
    di = dinv[...]
    h = jnp.maximum((p2[0] + p2[1] - yin[...]) * di + b[...], 0.0)
    y[...] = jnp.dot(h, w[...], preferred_element_type=jnp.float32,
                     precision=lax.Precision.HIGHEST) * di


_mid_call = pl.pallas_call(
    _mid_body,
    grid=(_NP // _BR,),
    in_specs=[
        pl.BlockSpec((_NC, _BR, _D), lambda i: (0, i, 0)),
        pl.BlockSpec((_BR, _D), lambda i: (i, 0)),
        pl.BlockSpec((_BR, 1), lambda i: (i, 0)),
        pl.BlockSpec((_D, _D), lambda i: (0, 0)),
        pl.BlockSpec((1, _D), lambda i: (0, 0)),
    ],
    out_specs=pl.BlockSpec((_BR, _D), lambda i: (i, 0)),
    out_shape=jax.ShapeDtypeStruct((_NP, _D), jnp.float32),
)


def _last_body(p, yin, dinv, b, h):
    p2 = p[...]
    h[...] = jnp.maximum((p2[0] + p2[1] - yin[...]) * dinv[...] + b[...], 0.0)


_last_call = pl.pallas_call(
    _last_body,
    grid=(_NP // _BR,),
    in_specs=[
        pl.BlockSpec((_NC, _BR, _D), lambda i: (0, i, 0)),
        pl.BlockSpec((_BR, _D), lambda i: (i, 0)),
        pl.BlockSpec((_BR, 1), lambda i: (i, 0)),
        pl.BlockSpec((1, _D), lambda i: (0, 0)),
    ],
    out_specs=pl.BlockSpec((_BR, _D), lambda i: (i, 0)),
    out_shape=jax.ShapeDtypeStruct((_NP, _D), jnp.float32),
)


def kernel(x, pos, batch, idx, edge_index, W1, b1, W2, b2, W3, b3):
    m = idx.shape[0]
    src = edge_index[0].astype(jnp.int32)
    dst = edge_index[1].astype(jnp.int32)
    srcf = jnp.pad(src, (0, _EP - _E)).reshape(_NW * _J, _C)
    dstp = jnp.pad(dst, (0, _EP - _E), constant_values=_N).reshape(_NW, _JW, 128)
    dstf = dstp.reshape(_NW * _J, _C)
    zeros128 = jnp.zeros((_ZR, _D), jnp.float32)
    ones128 = jnp.ones((_C, _D), jnp.float32)
    xp = jnp.pad(x, ((0, _NP - _N), (0, 0)))
    pos8 = jnp.pad(pos, ((0, _NP - _N), (0, 5)))
    w1x = W1[:_D]
    w1p = jnp.pad(W1[_D:], ((0, 5), (0, 0)))
    posb = jnp.concatenate(
        [pos, batch.astype(jnp.float32)[:, None],
         jnp.zeros((_N, _D - 4), jnp.float32)], axis=1)
    idxp = jnp.pad(idx.astype(jnp.int32), (0, _IP - m)).reshape(_NW, _IPW)
    b1r = b1.reshape(1, _D)
    b2r = b2.reshape(1, _D)
    b3r = b3.reshape(1, _D)

    deg = _deg_call(dstp, ones128, zeros128)
    y1, dinv = _stage1_call(deg, xp, pos8, w1x, w1p)
    p = _prop_call(y1, srcf, dstf)
    y2 = _mid_call(p, y1, dinv, W2, b1r)
    p = _prop_call(y2, srcf, dstf)
    y3 = _mid_call(p, y2, dinv, W3, b2r)
    p = _prop_call(y3, srcf, dstf)
    h3 = _last_call(p, y3, dinv, b3r)
    hout, pbout = _gat_call(h3, posb, idxp)
    return (
        hout[:m],
        pbout[:m, :3],
        pbout[:m, 3].astype(jnp.int32),
    )


# speedup vs baseline: 1.6056x; 1.2221x over previous
import functools

import jax
import jax.numpy as jnp
from jax import lax
from jax.experimental import pallas as pl
from jax.experimental.pallas import tpu as pltpu
from jax.experimental.pallas import tpu_sc as plsc

_N = 10000
_E = 320000
_D = 128
_NC = 2
_NS = 16
_NW = _NC * _NS
_C = 128
_J = 80
_EPW = _J * _C
_JW = _EPW // 128
_EP = _NW * _EPW
_NP = 10240
_RPT = _NP // _NS
_AR = _NP
_ZR = _RPT
_G = 2
_J0 = 120
_J1 = 40
_JMX = 120
_IPW = 80
_IP = _NW * _IPW

_MESH = plsc.VectorSubcoreMesh(
    core_axis_name="c", subcore_axis_name="s", num_cores=_NC, num_subcores=_NS
)


def _stage_idx(dst1d, src2d, g):
    r = g // (128 // _C)
    cb = (g % (128 // _C)) * _C
    for k in range(_C // 16):
        dst1d[pl.ds(k * 16, 16)] = src2d[r, pl.ds(cb + k * 16, 16)]


def _deg_body(dstp, ones_hbm, zeros_hbm, deg, dst_v, didx, ones_v, acc):
    c = lax.axis_index("c")
    s = lax.axis_index("s")
    wid = c * _NS + s
    pltpu.sync_copy(dstp.at[wid], dst_v)
    pltpu.sync_copy(ones_hbm, ones_v)
    pltpu.sync_copy(zeros_hbm, acc.at[pl.ds(s * _ZR, _ZR)])
    plsc.subcore_barrier()

    def body(j, carry):
        _stage_idx(didx, dst_v, j)
        pltpu.sync_copy(ones_v, acc.at[didx], add=True)
        return carry

    lax.fori_loop(0, _J, body, 0)
    plsc.subcore_barrier()
    pltpu.sync_copy(acc.at[pl.ds(s * _RPT, _RPT)], deg.at[c, pl.ds(s * _RPT, _RPT)])


_deg_call = pl.kernel(
    _deg_body,
    out_type=jax.ShapeDtypeStruct((_NC, _NP, _D), jnp.float32),
    mesh=_MESH,
    scratch_types=[
        pltpu.VMEM((_JW, 128), jnp.int32),
        pltpu.VMEM((_C,), jnp.int32),
        pltpu.VMEM((_C, _D), jnp.float32),
        pltpu.VMEM_SHARED((_AR, _D), jnp.float32),
    ],
)


def _prop_body(y_hbm, srcf, dstf, out,
               dst_v, sidx0, sidx1, didx0, didx1, rows0, rows1, acc,
               isem0, isem1, gsem0, gsem1, ssem0, ssem1):
    c = lax.axis_index("c")
    s = lax.axis_index("s")
    jn = lax.select(c == 0, _J0, _J1)
    base = lax.select(c == 0, s * _J0, _NS * _J0 + s * _J1)
    pltpu.sync_copy(dstf.at[pl.ds(base, _JMX)], dst_v)
    pltpu.sync_copy(y_hbm.at[pl.ds(s * _RPT, _RPT)], acc.at[pl.ds(s * _RPT, _RPT)])
    sidx = (sidx0, sidx1)
    didx = (didx0, didx1)
    rows = (rows0, rows1)
    isem = (isem0, isem1)
    gsem = (gsem0, gsem1)
    ssem = (ssem0, ssem1)
    for b in range(2):
        pltpu.async_copy(srcf.at[base + b], sidx[b], isem[b])
    plsc.subcore_barrier()

    def group(t, carry):
        for b in range(2):
            @pl.when(t > 0)
            def _():
                pltpu.make_async_copy(rows[b], acc.at[didx[b]], ssem[b]).wait()

            _stage_idx(didx[b], dst_v, 2 * t + b)
        for b in range(2):
            pltpu.make_async_copy(srcf.at[base + 2 * t + b], sidx[b], isem[b]).wait()
            pltpu.async_copy(y_hbm.at[sidx[b]], rows[b], gsem[b])
        for b in range(2):
            g = 2 * t + b
            pltpu.make_async_copy(y_hbm.at[sidx[b]], rows[b], gsem[b]).wait()
            pltpu.async_copy(rows[b], acc.at[didx[b]], ssem[b], add=True)

            @pl.when(g + 2 < jn)
            def _():
                pltpu.async_copy(srcf.at[base + g + 2], sidx[b], isem[b])

        return carry

    lax.fori_loop(0, jn // 2, group, 0)
    for b in range(2):
        pltpu.make_async_copy(rows[b], acc.at[didx[b]], ssem[b]).wait()
    plsc.subcore_barrier()
    pltpu.sync_copy(acc.at[pl.ds(s * _RPT, _RPT)], out.at[c, pl.ds(s * _RPT, _RPT)])


_prop_call = pl.kernel(
    _prop_body,
    out_type=jax.ShapeDtypeStruct((_NC, _NP, _D), jnp.float32),
    mesh=_MESH,
    scratch_types=(
        [pltpu.VMEM((_JMX, 128), jnp.int32)]
        + [pltpu.VMEM((_C,), jnp.int32)] * 4
        + [pltpu.VMEM((_C, _D), jnp.float32)] * 2
        + [pltpu.VMEM_SHARED((_AR, _D), jnp.float32)]
        + [pltpu.SemaphoreType.DMA] * 6
    ),
)


def _gat_body(h_hbm, pb_hbm, idxp, hout, pbout, idx_v, hrows, prows, sem):
    c = lax.axis_index("c")
    s = lax.axis_index("s")
    wid = c * _NS + s
    pltpu.sync_copy(idxp.at[wid], idx_v)
    pltpu.async_copy(h_hbm.at[idx_v], hrows, sem).wait()
    pltpu.async_copy(pb_hbm.at[idx_v], prows, sem).wait()
    pltpu.sync_copy(hrows, hout.at[pl.ds(wid * _IPW, _IPW)])
    pltpu.sync_copy(prows, pbout.at[pl.ds(wid * _IPW, _IPW)])


_gat_call = pl.kernel(
    _gat_body,
    out_type=(
        jax.ShapeDtypeStruct((_IP, _D), jnp.float32),
        jax.ShapeDtypeStruct((_IP, _D), jnp.float32),
    ),
    mesh=_MESH,
    scratch_types=[
        pltpu.VMEM((_IPW,), jnp.int32),
        pltpu.VMEM((_IPW, _D), jnp.float32),
        pltpu.VMEM((_IPW, _D), jnp.float32),
        pltpu.SemaphoreType.DMA,
    ],
)


_BR = 1024


def _stage1_body(deg, x, pos8, w1x, w1p, y, dinv):
    d2 = deg[...]
    dg = d2[0][:, 0:1] + d2[1][:, 0:1] + 1.0
    di = lax.rsqrt(dg)
    h = jnp.dot(x[...], w1x[...], preferred_element_type=jnp.float32,
                precision=lax.Precision.HIGHEST)
    h = h + jnp.dot(pos8[...], w1p[...], preferred_element_type=jnp.float32,
                    precision=lax.Precision.HIGHEST)
    y[...] = h * di
    dinv[...] = di


_stage1_call = pl.pallas_call(
    _stage1_body,
    grid=(_NP // _BR,),
    in_specs=[
        pl.BlockSpec((_NC, _BR, _D), lambda i: (0, i, 0)),
        pl.BlockSpec((_BR, _D), lambda i: (i, 0)),
        pl.BlockSpec((_BR, 8), lambda i: (i, 0)),
        pl.BlockSpec((_D, _D), lambda i: (0, 0)),
        pl.BlockSpec((8, _D), lambda i: (0, 0)),
    ],
    out_specs=[
        pl.BlockSpec((_BR, _D), lambda i: (i, 0)),
        pl.BlockSpec((_BR, 1), lambda i: (i, 0)),
    ],
    out_shape=[
        jax.ShapeDtypeStruct((_NP, _D), jnp.float32),
        jax.ShapeDtypeStruct((_NP, 1), jnp.float32),
    ],
)


def _mid_body(p, yin, dinv, w, b, y):
    p2 = p[...]
    di = dinv[...]
    h = jnp.maximum((p2[0] + p2[1] - yin[...]) * di + b[...], 0.0)
    y[...] = jnp.dot(h, w[...], preferred_element_type=jnp.float32,
                     precision=lax.Precision.HIGHEST) * di


_mid_call = pl.pallas_call(
    _mid_body,
    grid=(_NP // _BR,),
    in_specs=[
        pl.BlockSpec((_NC, _BR, _D), lambda i: (0, i, 0)),
        pl.BlockSpec((_BR, _D), lambda i: (i, 0)),
        pl.BlockSpec((_BR, 1), lambda i: (i, 0)),
        pl.BlockSpec((_D, _D), lambda i: (0, 0)),
        pl.BlockSpec((1, _D), lambda i: (0, 0)),
    ],
    out_specs=pl.BlockSpec((_BR, _D), lambda i: (i, 0)),
    out_shape=jax.ShapeDtypeStruct((_NP, _D), jnp.float32),
)


def _last_body(p, yin, dinv, b, h):
    p2 = p[...]
    h[...] = jnp.maximum((p2[0] + p2[1] - yin[...]) * dinv[...] + b[...], 0.0)


_last_call = pl.pallas_call(
    _last_body,
    grid=(_NP // _BR,),
    in_specs=[
        pl.BlockSpec((_NC, _BR, _D), lambda i: (0, i, 0)),
        pl.BlockSpec((_BR, _D), lambda i: (i, 0)),
        pl.BlockSpec((_BR, 1), lambda i: (i, 0)),
        pl.BlockSpec((1, _D), lambda i: (0, 0)),
    ],
    out_specs=pl.BlockSpec((_BR, _D), lambda i: (i, 0)),
    out_shape=jax.ShapeDtypeStruct((_NP, _D), jnp.float32),
)


def kernel(x, pos, batch, idx, edge_index, W1, b1, W2, b2, W3, b3):
    m = idx.shape[0]
    src = edge_index[0].astype(jnp.int32)
    dst = edge_index[1].astype(jnp.int32)
    srcf = jnp.pad(src, (0, _EP - _E)).reshape(_NW * _J, _C)
    dstp = jnp.pad(dst, (0, _EP - _E), constant_values=_N).reshape(_NW, _JW, 128)
    dstf = dstp.reshape(_NW * _J, _C)
    zeros128 = jnp.zeros((_ZR, _D), jnp.float32)
    ones128 = jnp.ones((_C, _D), jnp.float32)
    xp = jnp.pad(x, ((0, _NP - _N), (0, 0)))
    pos8 = jnp.pad(pos, ((0, _NP - _N), (0, 5)))
    w1x = W1[:_D]
    w1p = jnp.pad(W1[_D:], ((0, 5), (0, 0)))
    posb = jnp.concatenate(
        [pos, batch.astype(jnp.float32)[:, None],
         jnp.zeros((_N, _D - 4), jnp.float32)], axis=1)
    idxp = jnp.pad(idx.astype(jnp.int32), (0, _IP - m)).reshape(_NW, _IPW)
    b1r = b1.reshape(1, _D)
    b2r = b2.reshape(1, _D)
    b3r = b3.reshape(1, _D)

    deg = _deg_call(dstp, ones128, zeros128)
    y1, dinv = _stage1_call(deg, xp, pos8, w1x, w1p)
    p = _prop_call(y1, srcf, dstf)
    y2 = _mid_call(p, y1, dinv, W2, b1r)
    p = _prop_call(y2, srcf, dstf)
    y3 = _mid_call(p, y2, dinv, W3, b2r)
    p = _prop_call(y3, srcf, dstf)
    h3 = _last_call(p, y3, dinv, b3r)
    hout, pbout = _gat_call(h3, posb, idxp)
    return (
        hout[:m],
        pbout[:m, :3],
        pbout[:m, 3].astype(jnp.int32),
    )


# speedup vs baseline: 1.6435x; 1.0236x over previous
import functools

import jax
import jax.numpy as jnp
from jax import lax
from jax.experimental import pallas as pl
from jax.experimental.pallas import tpu as pltpu
from jax.experimental.pallas import tpu_sc as plsc

_N = 10000
_E = 320000
_D = 128
_NC = 2
_NS = 16
_NW = _NC * _NS
_C = 128
_J = 80
_EPW = _J * _C
_JW = _EPW // 128
_EP = _NW * _EPW
_NP = 10240
_RPT = _NP // _NS
_AR = 10112
_ART = _AR // _NS
_ZR = _RPT
_G = 2
_J0 = 128
_J1 = 32
_JMX = 128
_IPW = 80
_IP = _NW * _IPW

_MESH = plsc.VectorSubcoreMesh(
    core_axis_name="c", subcore_axis_name="s", num_cores=_NC, num_subcores=_NS
)


def _stage_idx(dst1d, src2d, g):
    r = g // (128 // _C)
    cb = (g % (128 // _C)) * _C
    for k in range(_C // 16):
        dst1d[pl.ds(k * 16, 16)] = src2d[r, pl.ds(cb + k * 16, 16)]


def _deg_body(dstp, ones_hbm, zeros_hbm, deg, dst_v, didx, ones_v, acc):
    c = lax.axis_index("c")
    s = lax.axis_index("s")
    wid = c * _NS + s
    pltpu.sync_copy(dstp.at[wid], dst_v)
    pltpu.sync_copy(ones_hbm, ones_v)
    pltpu.sync_copy(zeros_hbm, acc.at[pl.ds(s * _ZR, _ZR)])
    plsc.subcore_barrier()

    def body(j, carry):
        _stage_idx(didx, dst_v, j)
        pltpu.sync_copy(ones_v, acc.at[didx], add=True)
        return carry

    lax.fori_loop(0, _J, body, 0)
    plsc.subcore_barrier()
    pltpu.sync_copy(acc.at[pl.ds(s * _RPT, _RPT)], deg.at[c, pl.ds(s * _RPT, _RPT)])


_deg_call = pl.kernel(
    _deg_body,
    out_type=jax.ShapeDtypeStruct((_NC, _NP, _D), jnp.float32),
    mesh=_MESH,
    scratch_types=[
        pltpu.VMEM((_JW, 128), jnp.int32),
        pltpu.VMEM((_C,), jnp.int32),
        pltpu.VMEM((_C, _D), jnp.float32),
        pltpu.VMEM_SHARED((_NP, _D), jnp.float32),
    ],
)


def _prop_body(y_hbm, srcf, dstf, out,
               dst_v, sidx0, sidx1, didx0, didx1, rows0, rows1, acc,
               isem0, isem1, gsem0, gsem1, ssem0, ssem1):
    c = lax.axis_index("c")
    s = lax.axis_index("s")
    jn = lax.select(c == 0, _J0, _J1)
    base = lax.select(c == 0, s * _J0, _NS * _J0 + s * _J1)
    pltpu.sync_copy(dstf.at[pl.ds(base, _JMX)], dst_v)
    pltpu.sync_copy(y_hbm.at[pl.ds(s * _ART, _ART)], acc.at[pl.ds(s * _ART, _ART)])
    sidx = (sidx0, sidx1)
    didx = (didx0, didx1)
    rows = (rows0, rows1)
    isem = (isem0, isem1)
    gsem = (gsem0, gsem1)
    ssem = (ssem0, ssem1)
    for b in range(2):
        pltpu.async_copy(srcf.at[base + b], sidx[b], isem[b])
    plsc.subcore_barrier()

    def group(t, carry):
        for b in range(2):
            @pl.when(t > 0)
            def _():
                pltpu.make_async_copy(rows[b], acc.at[didx[b]], ssem[b]).wait()

            _stage_idx(didx[b], dst_v, 2 * t + b)
        for b in range(2):
            pltpu.make_async_copy(srcf.at[base + 2 * t + b], sidx[b], isem[b]).wait()
            pltpu.async_copy(y_hbm.at[sidx[b]], rows[b], gsem[b])
        for b in range(2):
            g = 2 * t + b
            pltpu.make_async_copy(y_hbm.at[sidx[b]], rows[b], gsem[b]).wait()
            pltpu.async_copy(rows[b], acc.at[didx[b]], ssem[b], add=True)

            @pl.when(g + 2 < jn)
            def _():
                pltpu.async_copy(srcf.at[base + g + 2], sidx[b], isem[b])

        return carry

    lax.fori_loop(0, jn // 2, group, 0)
    for b in range(2):
        pltpu.make_async_copy(rows[b], acc.at[didx[b]], ssem[b]).wait()
    plsc.subcore_barrier()
    pltpu.sync_copy(acc.at[pl.ds(s * _ART, _ART)], out.at[c, pl.ds(s * _ART, _ART)])


_prop_call = pl.kernel(
    _prop_body,
    out_type=jax.ShapeDtypeStruct((_NC, _NP, _D), jnp.float32),
    mesh=_MESH,
    scratch_types=(
        [pltpu.VMEM((_JMX, 128), jnp.int32)]
        + [pltpu.VMEM((_C,), jnp.int32)] * 4
        + [pltpu.VMEM((_C, _D), jnp.float32)] * 2
        + [pltpu.VMEM_SHARED((_AR, _D), jnp.float32)]
        + [pltpu.SemaphoreType.DMA] * 6
    ),
)


def _gat_body(h_hbm, pb_hbm, idxp, hout, pbout, idx_v, hrows, prows, sem):
    c = lax.axis_index("c")
    s = lax.axis_index("s")
    wid = c * _NS + s
    pltpu.sync_copy(idxp.at[wid], idx_v)
    pltpu.async_copy(h_hbm.at[idx_v], hrows, sem).wait()
    pltpu.async_copy(pb_hbm.at[idx_v], prows, sem).wait()
    pltpu.sync_copy(hrows, hout.at[pl.ds(wid * _IPW, _IPW)])
    pltpu.sync_copy(prows, pbout.at[pl.ds(wid * _IPW, _IPW)])


_gat_call = pl.kernel(
    _gat_body,
    out_type=(
        jax.ShapeDtypeStruct((_IP, _D), jnp.float32),
        jax.ShapeDtypeStruct((_IP, _D), jnp.float32),
    ),
    mesh=_MESH,
    scratch_types=[
        pltpu.VMEM((_IPW,), jnp.int32),
        pltpu.VMEM((_IPW, _D), jnp.float32),
        pltpu.VMEM((_IPW, _D), jnp.float32),
        pltpu.SemaphoreType.DMA,
    ],
)


_BR = 1024


def _stage1_body(deg, x, pos8, w1x, w1p, y, dinv):
    d2 = deg[...]
    dg = d2[0][:, 0:1] + d2[1][:, 0:1] + 1.0
    di = lax.rsqrt(dg)
    h = jnp.dot(x[...], w1x[...], preferred_element_type=jnp.float32,
                precision=lax.Precision.HIGHEST)
    h = h + jnp.dot(pos8[...], w1p[...], preferred_element_type=jnp.float32,
                    precision=lax.Precision.HIGHEST)
    y[...] = h * di
    dinv[...] = di


_stage1_call = pl.pallas_call(
    _stage1_body,
    grid=(_NP // _BR,),
    in_specs=[
        pl.BlockSpec((_NC, _BR, _D), lambda i: (0, i, 0)),
        pl.BlockSpec((_BR, _D), lambda i: (i, 0)),
        pl.BlockSpec((_BR, 8), lambda i: (i, 0)),
        pl.BlockSpec((_D, _D), lambda i: (0, 0)),
        pl.BlockSpec((8, _D), lambda i: (0, 0)),
    ],
    out_specs=[
        pl.BlockSpec((_BR, _D), lambda i: (i, 0)),
        pl.BlockSpec((_BR, 1), lambda i: (i, 0)),
    ],
    out_shape=[
        jax.ShapeDtypeStruct((_NP, _D), jnp.float32),
        jax.ShapeDtypeStruct((_NP, 1), jnp.float32),
    ],
)


def _mid_body(p, yin, dinv, w, b, y):
    p2 = p[...]
    di = dinv[...]
    h = jnp.maximum((p2[0] + p2[1] - yin[...]) * di + b[...], 0.0)
    y[...] = jnp.dot(h, w[...], preferred_element_type=jnp.float32,
                     precision=lax.Precision.HIGHEST) * di


_mid_call = pl.pallas_call(
    _mid_body,
    grid=(_NP // _BR,),
    in_specs=[
        pl.BlockSpec((_NC, _BR, _D), lambda i: (0, i, 0)),
        pl.BlockSpec((_BR, _D), lambda i: (i, 0)),
        pl.BlockSpec((_BR, 1), lambda i: (i, 0)),
        pl.BlockSpec((_D, _D), lambda i: (0, 0)),
        pl.BlockSpec((1, _D), lambda i: (0, 0)),
    ],
    out_specs=pl.BlockSpec((_BR, _D), lambda i: (i, 0)),
    out_shape=jax.ShapeDtypeStruct((_NP, _D), jnp.float32),
)


def _last_body(p, yin, dinv, b, h):
    p2 = p[...]
    h[...] = jnp.maximum((p2[0] + p2[1] - yin[...]) * dinv[...] + b[...], 0.0)


_last_call = pl.pallas_call(
    _last_body,
    grid=(_NP // _BR,),
    in_specs=[
        pl.BlockSpec((_NC, _BR, _D), lambda i: (0, i, 0)),
        pl.BlockSpec((_BR, _D), lambda i: (i, 0)),
        pl.BlockSpec((_BR, 1), lambda i: (i, 0)),
        pl.BlockSpec((1, _D), lambda i: (0, 0)),
    ],
    out_specs=pl.BlockSpec((_BR, _D), lambda i: (i, 0)),
    out_shape=jax.ShapeDtypeStruct((_NP, _D), jnp.float32),
)


def kernel(x, pos, batch, idx, edge_index, W1, b1, W2, b2, W3, b3):
    m = idx.shape[0]
    src = edge_index[0].astype(jnp.int32)
    dst = edge_index[1].astype(jnp.int32)
    srcf = jnp.pad(src, (0, _EP - _E)).reshape(_NW * _J, _C)
    dstp = jnp.pad(dst, (0, _EP - _E), constant_values=_N).reshape(_NW, _JW, 128)
    dstf = dstp.reshape(_NW * _J, _C)
    zeros128 = jnp.zeros((_ZR, _D), jnp.float32)
    ones128 = jnp.ones((_C, _D), jnp.float32)
    xp = jnp.pad(x, ((0, _NP - _N), (0, 0)))
    pos8 = jnp.pad(pos, ((0, _NP - _N), (0, 5)))
    w1x = W1[:_D]
    w1p = jnp.pad(W1[_D:], ((0, 5), (0, 0)))
    posb = jnp.concatenate(
        [pos, batch.astype(jnp.float32)[:, None],
         jnp.zeros((_N, _D - 4), jnp.float32)], axis=1)
    idxp = jnp.pad(idx.astype(jnp.int32), (0, _IP - m)).reshape(_NW, _IPW)
    b1r = b1.reshape(1, _D)
    b2r = b2.reshape(1, _D)
    b3r = b3.reshape(1, _D)

    deg = _deg_call(dstp, ones128, zeros128)
    y1, dinv = _stage1_call(deg, xp, pos8, w1x, w1p)
    p = _prop_call(y1, srcf, dstf)
    y2 = _mid_call(p, y1, dinv, W2, b1r)
    p = _prop_call(y2, srcf, dstf)
    y3 = _mid_call(p, y2, dinv, W3, b2r)
    p = _prop_call(y3, srcf, dstf)
    h3 = _last_call(p, y3, dinv, b3r)
    hout, pbout = _gat_call(h3, posb, idxp)
    return (
        hout[:m],
        pbout[:m, :3],
        pbout[:m, 3].astype(jnp.int32),
    )


# speedup vs baseline: 1.6444x; 1.0006x over previous
import jax
import jax.numpy as jnp
from jax import lax
from jax.experimental import pallas as pl
from jax.experimental.pallas import tpu as pltpu
from jax.experimental.pallas import tpu_sc as plsc

_N = 10000
_E = 320000
_D = 128
_NC = 2
_NS = 16
_NW = _NC * _NS
_C = 128
_J = 80
_EPW = _J * _C
_JW = _EPW // 128
_EP = _NW * _EPW
_NP = 10240
_RPT = _NP // _NS
_AR = 10112
_ART = _AR // _NS
_ZR = _RPT
_G = 2
_J0 = 128
_J1 = 32
_JMX = 128
_IPW = 80
_IP = _NW * _IPW

_MESH = plsc.VectorSubcoreMesh(
    core_axis_name="c", subcore_axis_name="s", num_cores=_NC, num_subcores=_NS
)


def _stage_idx(dst1d, src2d, g):
    r = g // (128 // _C)
    cb = (g % (128 // _C)) * _C
    for k in range(_C // 16):
        dst1d[pl.ds(k * 16, 16)] = src2d[r, pl.ds(cb + k * 16, 16)]


def _deg_body(dstp, ones_hbm, zeros_hbm, deg, dst_v, didx, ones_v, acc):
    c = lax.axis_index("c")
    s = lax.axis_index("s")
    wid = c * _NS + s
    pltpu.sync_copy(dstp.at[wid], dst_v)
    pltpu.sync_copy(ones_hbm, ones_v)
    pltpu.sync_copy(zeros_hbm, acc.at[pl.ds(s * _ZR, _ZR)])
    plsc.subcore_barrier()

    def body(j, carry):
        _stage_idx(didx, dst_v, j)
        pltpu.sync_copy(ones_v, acc.at[didx], add=True)
        return carry

    lax.fori_loop(0, _J, body, 0)
    plsc.subcore_barrier()
    pltpu.sync_copy(acc.at[pl.ds(s * _RPT, _RPT)], deg.at[c, pl.ds(s * _RPT, _RPT)])


_deg_call = pl.kernel(
    _deg_body,
    out_type=jax.ShapeDtypeStruct((_NC, _NP, _D), jnp.float32),
    mesh=_MESH,
    scratch_types=[
        pltpu.VMEM((_JW, 128), jnp.int32),
        pltpu.VMEM((_C,), jnp.int32),
        pltpu.VMEM((_C, _D), jnp.float32),
        pltpu.VMEM_SHARED((_NP, _D), jnp.float32),
    ],
)


def _prop_body(y_hbm, srcf, dstf, out,
               dst_v, sidx0, sidx1, didx0, didx1, rows0, rows1, acc,
               isem0, isem1, gsem0, gsem1, ssem0, ssem1):
    c = lax.axis_index("c")
    s = lax.axis_index("s")
    jn = lax.select(c == 0, _J0, _J1)
    base = lax.select(c == 0, s * _J0, _NS * _J0 + s * _J1)
    pltpu.sync_copy(dstf.at[pl.ds(base, _JMX)], dst_v)
    pltpu.sync_copy(y_hbm.at[pl.ds(s * _ART, _ART)], acc.at[pl.ds(s * _ART, _ART)])
    sidx = (sidx0, sidx1)
    didx = (didx0, didx1)
    rows = (rows0, rows1)
    isem = (isem0, isem1)
    gsem = (gsem0, gsem1)
    ssem = (ssem0, ssem1)
    for b in range(2):
        pltpu.async_copy(srcf.at[base + b], sidx[b], isem[b])
    plsc.subcore_barrier()

    def group(t, carry):
        for b in range(2):
            @pl.when(t > 0)
            def _():
                pltpu.make_async_copy(rows[b], acc.at[didx[b]], ssem[b]).wait()

            _stage_idx(didx[b], dst_v, 2 * t + b)
        for b in range(2):
            pltpu.make_async_copy(srcf.at[base + 2 * t + b], sidx[b], isem[b]).wait()
            pltpu.async_copy(y_hbm.at[sidx[b]], rows[b], gsem[b])
        for b in range(2):
            g = 2 * t + b
            pltpu.make_async_copy(y_hbm.at[sidx[b]], rows[b], gsem[b]).wait()
            pltpu.async_copy(rows[b], acc.at[didx[b]], ssem[b], add=True)

            @pl.when(g + 2 < jn)
            def _():
                pltpu.async_copy(srcf.at[base + g + 2], sidx[b], isem[b])

        return carry

    lax.fori_loop(0, jn // 2, group, 0)
    for b in range(2):
        pltpu.make_async_copy(rows[b], acc.at[didx[b]], ssem[b]).wait()
    plsc.subcore_barrier()
    pltpu.sync_copy(acc.at[pl.ds(s * _ART, _ART)], out.at[c, pl.ds(s * _ART, _ART)])


_prop_call = pl.kernel(
    _prop_body,
    out_type=jax.ShapeDtypeStruct((_NC, _NP, _D), jnp.float32),
    mesh=_MESH,
    scratch_types=(
        [pltpu.VMEM((_JMX, 128), jnp.int32)]
        + [pltpu.VMEM((_C,), jnp.int32)] * 4
        + [pltpu.VMEM((_C, _D), jnp.float32)] * 2
        + [pltpu.VMEM_SHARED((_AR, _D), jnp.float32)]
        + [pltpu.SemaphoreType.DMA] * 6
    ),
)


def _gat_body(h_hbm, pb_hbm, idxp, hout, pbout, idx_v, hrows, prows, sem):
    c = lax.axis_index("c")
    s = lax.axis_index("s")
    wid = c * _NS + s
    pltpu.sync_copy(idxp.at[wid], idx_v)
    pltpu.async_copy(h_hbm.at[idx_v], hrows, sem).wait()
    pltpu.async_copy(pb_hbm.at[idx_v], prows, sem).wait()
    pltpu.sync_copy(hrows, hout.at[pl.ds(wid * _IPW, _IPW)])
    pltpu.sync_copy(prows, pbout.at[pl.ds(wid * _IPW, _IPW)])


_gat_call = pl.kernel(
    _gat_body,
    out_type=(
        jax.ShapeDtypeStruct((_IP, _D), jnp.float32),
        jax.ShapeDtypeStruct((_IP, _D), jnp.float32),
    ),
    mesh=_MESH,
    scratch_types=[
        pltpu.VMEM((_IPW,), jnp.int32),
        pltpu.VMEM((_IPW, _D), jnp.float32),
        pltpu.VMEM((_IPW, _D), jnp.float32),
        pltpu.SemaphoreType.DMA,
    ],
)


_BR = 1024


def _stage1_body(deg, x, pos8, w1x, w1p, y, dinv):
    d2 = deg[...]
    dg = d2[0][:, 0:1] + d2[1][:, 0:1] + 1.0
    di = lax.rsqrt(dg)
    h = jnp.dot(x[...], w1x[...], preferred_element_type=jnp.float32,
                precision=lax.Precision.HIGHEST)
    h = h + jnp.dot(pos8[...], w1p[...], preferred_element_type=jnp.float32,
                    precision=lax.Precision.HIGHEST)
    y[...] = h * di
    dinv[...] = di


_stage1_call = pl.pallas_call(
    _stage1_body,
    grid=(_NP // _BR,),
    in_specs=[
        pl.BlockSpec((_NC, _BR, _D), lambda i: (0, i, 0)),
        pl.BlockSpec((_BR, _D), lambda i: (i, 0)),
        pl.BlockSpec((_BR, 8), lambda i: (i, 0)),
        pl.BlockSpec((_D, _D), lambda i: (0, 0)),
        pl.BlockSpec((8, _D), lambda i: (0, 0)),
    ],
    out_specs=[
        pl.BlockSpec((_BR, _D), lambda i: (i, 0)),
        pl.BlockSpec((_BR, 1), lambda i: (i, 0)),
    ],
    out_shape=[
        jax.ShapeDtypeStruct((_NP, _D), jnp.float32),
        jax.ShapeDtypeStruct((_NP, 1), jnp.float32),
    ],
)


def _mid_body(p, yin, dinv, w, b, y):
    p2 = p[...]
    di = dinv[...]
    h = jnp.maximum((p2[0] + p2[1] - yin[...]) * di + b[...], 0.0)
    y[...] = jnp.dot(h, w[...], preferred_element_type=jnp.float32,
                     precision=lax.Precision.HIGHEST) * di


_mid_call = pl.pallas_call(
    _mid_body,
    grid=(_NP // _BR,),
    in_specs=[
        pl.BlockSpec((_NC, _BR, _D), lambda i: (0, i, 0)),
        pl.BlockSpec((_BR, _D), lambda i: (i, 0)),
        pl.BlockSpec((_BR, 1), lambda i: (i, 0)),
        pl.BlockSpec((_D, _D), lambda i: (0, 0)),
        pl.BlockSpec((1, _D), lambda i: (0, 0)),
    ],
    out_specs=pl.BlockSpec((_BR, _D), lambda i: (i, 0)),
    out_shape=jax.ShapeDtypeStruct((_NP, _D), jnp.float32),
)


def _last_body(p, yin, dinv, b, h):
    p2 = p[...]
    h[...] = jnp.maximum((p2[0] + p2[1] - yin[...]) * dinv[...] + b[...], 0.0)


_last_call = pl.pallas_call(
    _last_body,
    grid=(_NP // _BR,),
    in_specs=[
        pl.BlockSpec((_NC, _BR, _D), lambda i: (0, i, 0)),
        pl.BlockSpec((_BR, _D), lambda i: (i, 0)),
        pl.BlockSpec((_BR, 1), lambda i: (i, 0)),
        pl.BlockSpec((1, _D), lambda i: (0, 0)),
    ],
    out_specs=pl.BlockSpec((_BR, _D), lambda i: (i, 0)),
    out_shape=jax.ShapeDtypeStruct((_NP, _D), jnp.float32),
)


def kernel(x, pos, batch, idx, edge_index, W1, b1, W2, b2, W3, b3):
    m = idx.shape[0]
    src = edge_index[0].astype(jnp.int32)
    dst = edge_index[1].astype(jnp.int32)
    srcf = jnp.pad(src, (0, _EP - _E)).reshape(_NW * _J, _C)
    dstp = jnp.pad(dst, (0, _EP - _E), constant_values=_N).reshape(_NW, _JW, 128)
    dstf = dstp.reshape(_NW * _J, _C)
    zeros128 = jnp.zeros((_ZR, _D), jnp.float32)
    ones128 = jnp.ones((_C, _D), jnp.float32)
    xp = jnp.pad(x, ((0, _NP - _N), (0, 0)))
    pos8 = jnp.pad(pos, ((0, _NP - _N), (0, 5)))
    w1x = W1[:_D]
    w1p = jnp.pad(W1[_D:], ((0, 5), (0, 0)))
    posb = jnp.concatenate(
        [pos, batch.astype(jnp.float32)[:, None],
         jnp.zeros((_N, _D - 4), jnp.float32)], axis=1)
    idxp = jnp.pad(idx.astype(jnp.int32), (0, _IP - m)).reshape(_NW, _IPW)
    b1r = b1.reshape(1, _D)
    b2r = b2.reshape(1, _D)
    b3r = b3.reshape(1, _D)

    deg = _deg_call(dstp, ones128, zeros128)
    y1, dinv = _stage1_call(deg, xp, pos8, w1x, w1p)
    p = _prop_call(y1, srcf, dstf)
    y2 = _mid_call(p, y1, dinv, W2, b1r)
    p = _prop_call(y2, srcf, dstf)
    y3 = _mid_call(p, y2, dinv, W3, b2r)
    p = _prop_call(y3, srcf, dstf)
    h3 = _last_call(p, y3, dinv, b3r)
    hout, pbout = _gat_call(h3, posb, idxp)
    return (
        hout[:m],
        pbout[:m, :3],
        pbout[:m, 3].astype(jnp.int32),
    )
